# Initial kernel scaffold; baseline (speedup 1.0000x reference)
import jax
import jax.numpy as jnp
from jax.experimental import pallas as pl


def kernel(batch, table):
    raise NotImplementedError("write your pallas kernel here")


# speedup vs baseline: 1.5471x; 1.5471x over previous
import functools

import jax
import jax.numpy as jnp
from jax import lax
from jax.experimental import pallas as pl
from jax.experimental.pallas import tpu as pltpu
from jax.experimental.pallas import tpu_sc as plsc

_NUM_CORES = 2
_NUM_SUBCORES = 16
_NW = _NUM_CORES * _NUM_SUBCORES


@functools.partial(jax.jit, static_argnames=("chunk",))
def _sc_gather(table, idx, chunk=1024):
    B = idx.shape[0]
    D = table.shape[1]
    b_per_w = B // _NW
    n_chunks = b_per_w // chunk
    assert b_per_w % chunk == 0 and B % _NW == 0

    mesh = plsc.VectorSubcoreMesh(core_axis_name="c", subcore_axis_name="s")

    @functools.partial(
        pl.kernel,
        mesh=mesh,
        out_type=jax.ShapeDtypeStruct((B, D), jnp.float32),
        scratch_types=[
            pltpu.VMEM((chunk,), jnp.int32),
            pltpu.VMEM((chunk, D), jnp.float32),
            pltpu.SemaphoreType.DMA,
        ],
        compiler_params=pltpu.CompilerParams(use_tc_tiling_on_sc=False),
    )
    def k(table_hbm, idx_hbm, out_hbm, idx_v, rows_v, sem):
        wid = lax.axis_index("s") * _NUM_CORES + lax.axis_index("c")
        base = wid * b_per_w
        for g in range(n_chunks):
            off = base + g * chunk
            pltpu.sync_copy(idx_hbm.at[pl.ds(off, chunk)], idx_v)
            pltpu.async_copy(table_hbm.at[idx_v], rows_v, sem).wait()
            pltpu.sync_copy(rows_v, out_hbm.at[pl.ds(off, chunk)])

    return k(table, idx)


def kernel(batch, table):
    idx = batch.reshape(-1).astype(jnp.int32)
    out = _sc_gather(table, idx)
    return out.reshape(*batch.shape, table.shape[1])


# speedup vs baseline: 1.5757x; 1.0185x over previous
import functools

import jax
import jax.numpy as jnp
from jax import lax
from jax.experimental import pallas as pl
from jax.experimental.pallas import tpu as pltpu
from jax.experimental.pallas import tpu_sc as plsc

_NUM_CORES = 2
_NUM_SUBCORES = 16
_NW = _NUM_CORES * _NUM_SUBCORES


@functools.partial(jax.jit, static_argnames=("chunk",))
def _sc_gather(table, idx, chunk=1664):
    B = idx.shape[0]
    D = table.shape[1]
    b_per_w = B // _NW
    n = b_per_w // chunk
    assert b_per_w % chunk == 0 and B % _NW == 0 and chunk % 8 == 0

    mesh = plsc.VectorSubcoreMesh(core_axis_name="c", subcore_axis_name="s")

    @functools.partial(
        pl.kernel,
        mesh=mesh,
        out_type=jax.ShapeDtypeStruct((B, D), jnp.float32),
        scratch_types=[
            pltpu.VMEM((b_per_w,), jnp.int32),
            pltpu.VMEM((chunk, D), jnp.float32),
            pltpu.VMEM((chunk, D), jnp.float32),
            pltpu.SemaphoreType.DMA,
            pltpu.SemaphoreType.DMA,
            pltpu.SemaphoreType.DMA,
            pltpu.SemaphoreType.DMA,
        ],
        compiler_params=pltpu.CompilerParams(use_tc_tiling_on_sc=False),
    )
    def k(table_hbm, idx_hbm, out_hbm, idx_v, rows0, rows1, gs0, gs1, os0, os1):
        wid = lax.axis_index("s") * _NUM_CORES + lax.axis_index("c")
        base = wid * b_per_w
        rows = (rows0, rows1)
        gsem = (gs0, gs1)
        osem = (os0, os1)
        pltpu.sync_copy(idx_hbm.at[pl.ds(base, b_per_w)], idx_v)

        def gather(g):
            return pltpu.async_copy(
                table_hbm.at[idx_v.at[pl.ds(g * chunk, chunk)]],
                rows[g & 1], gsem[g & 1])

        def put(g):
            return pltpu.async_copy(
                rows[g & 1], out_hbm.at[pl.ds(base + g * chunk, chunk)],
                osem[g & 1])

        g_cp = [None] * n
        o_cp = [None] * n
        g_cp[0] = gather(0)
        for g in range(n):
            if g + 1 < n:
                if g >= 1:
                    o_cp[g - 1].wait()
                g_cp[g + 1] = gather(g + 1)
            g_cp[g].wait()
            o_cp[g] = put(g)
        if n >= 2:
            o_cp[n - 2].wait()
        o_cp[n - 1].wait()

    return k(table, idx)


def kernel(batch, table):
    idx = batch.reshape(-1).astype(jnp.int32)
    out = _sc_gather(table, idx)
    return out.reshape(*batch.shape, table.shape[1])


# speedup vs baseline: 1.6717x; 1.0609x over previous
import functools

import jax
import jax.numpy as jnp
from jax import lax
from jax.experimental import pallas as pl
from jax.experimental.pallas import tpu as pltpu
from jax.experimental.pallas import tpu_sc as plsc

_NUM_CORES = 2
_NUM_SUBCORES = 16
_NW = _NUM_CORES * _NUM_SUBCORES


@functools.partial(jax.jit, static_argnames=("chunk",))
def _sc_gather(table, idx, chunk=1664):
    B = idx.shape[0]
    D = table.shape[1]
    b_per_w = B // _NW
    n = b_per_w // chunk
    assert b_per_w % chunk == 0 and B % _NW == 0 and chunk % 8 == 0

    mesh = plsc.VectorSubcoreMesh(core_axis_name="c", subcore_axis_name="s")

    @functools.partial(
        pl.kernel,
        mesh=mesh,
        out_type=jax.ShapeDtypeStruct((B, D), jnp.float32),
        scratch_types=[
            pltpu.VMEM((b_per_w,), jnp.int32),
            pltpu.VMEM((chunk, D), jnp.float32),
            pltpu.VMEM((chunk, D), jnp.float32),
            pltpu.SemaphoreType.DMA,
            pltpu.SemaphoreType.DMA,
            pltpu.SemaphoreType.DMA,
            pltpu.SemaphoreType.DMA,
        ],
        compiler_params=pltpu.CompilerParams(use_tc_tiling_on_sc=False),
    )
    def k(table_hbm, idx_hbm, out_hbm, idx_v, rows0, rows1, gs0, gs1, os0, os1):
        wid = lax.axis_index("s") * _NUM_CORES + lax.axis_index("c")
        base = wid * b_per_w
        rows = (rows0, rows1)
        gsem = (gs0, gs1)
        osem = (os0, os1)
        pltpu.sync_copy(idx_hbm.at[pl.ds(base, b_per_w)], idx_v)

        def gather(g):
            return pltpu.async_copy(
                table_hbm.at[idx_v.at[pl.ds(g * chunk, chunk)]],
                rows[g & 1], gsem[g & 1])

        def put(g):
            return pltpu.async_copy(
                rows[g & 1], out_hbm.at[pl.ds(base + g * chunk, chunk)],
                osem[g & 1])

        g_cp = [None] * n
        o_cp = [None] * n
        g_cp[0] = gather(0)
        for g in range(n):
            if g + 1 < n:
                if g >= 1:
                    o_cp[g - 1].wait()
                g_cp[g + 1] = gather(g + 1)
            g_cp[g].wait()
            o_cp[g] = put(g)
        if n >= 2:
            o_cp[n - 2].wait()
        o_cp[n - 1].wait()

    return k(table, idx)


def kernel(batch, table):
    n, c = batch.shape
    d = table.shape[1]
    idx = batch.T.reshape(-1).astype(jnp.int32)
    out = _sc_gather(table, idx)
    return out.reshape(c, n, d).transpose(1, 0, 2)
